# Initial kernel scaffold; baseline (speedup 1.0000x reference)
import jax
import jax.numpy as jnp
from jax.experimental import pallas as pl


def kernel(x, edge_index, edge_feature, W_l0, b_l0, W_r0, W_l1, b_l1, W_r1):
    raise NotImplementedError("write your pallas kernel here")


# speedup vs baseline: 3.3204x; 3.3204x over previous
import functools

import jax
import jax.numpy as jnp
from jax import lax
from jax.experimental import pallas as pl
from jax.experimental.pallas import tpu as pltpu
from jax.experimental.pallas import tpu_sc as plsc

NC = 2
NS = 16
CHUNK = 80


def _segment_sum_sc(feat2, src2, dst, n):
    dh = feat2.shape[1]
    e = dst.shape[0]
    per_tile = e // NS
    nchunk = per_tile // CHUNK
    zr = -(-n // (NS * 8)) * 8
    n_pad = zr * NS

    mesh = plsc.VectorSubcoreMesh(core_axis_name="c", subcore_axis_name="s")

    @functools.partial(
        pl.kernel,
        out_type=jax.ShapeDtypeStruct((NC, n_pad, dh), jnp.float32),
        mesh=mesh,
        scratch_types=[
            pltpu.VMEM((CHUNK,), jnp.int32),
            pltpu.VMEM((CHUNK,), jnp.int32),
            pltpu.VMEM((CHUNK, dh), jnp.float32),
            pltpu.VMEM((zr, dh), jnp.float32),
            pltpu.VMEM_SHARED((n_pad, dh), jnp.float32),
            pltpu.SemaphoreType.DMA,
        ],
        compiler_params=pltpu.CompilerParams(use_tc_tiling_on_sc=False),
    )
    def seg(feat_hbm, src_hbm, dst_hbm, out_hbm,
            src_v, dst_v, rows_v, buf_v, acc_sh, sem):
        c = lax.axis_index("c")
        s = lax.axis_index("s")

        def zrow(r, carry):
            for j in range(dh // 16):
                buf_v[r, pl.ds(j * 16, 16)] = jnp.zeros((16,), jnp.float32)
            return carry
        lax.fori_loop(0, zr, zrow, 0)
        pltpu.sync_copy(buf_v, acc_sh.at[pl.ds(s * zr, zr)])
        plsc.subcore_barrier()

        base = s * per_tile

        def chunk(g, carry):
            off = base + g * CHUNK
            pltpu.sync_copy(src_hbm.at[pl.ds(c * e + off, CHUNK)], src_v)
            pltpu.sync_copy(dst_hbm.at[pl.ds(off, CHUNK)], dst_v)
            pltpu.async_copy(feat_hbm.at[src_v], rows_v, sem).wait()
            pltpu.sync_copy(rows_v, acc_sh.at[dst_v], add=True)
            return carry
        lax.fori_loop(0, nchunk, chunk, 0)

        plsc.subcore_barrier()
        pltpu.sync_copy(acc_sh.at[pl.ds(s * zr, zr)], buf_v)
        pltpu.sync_copy(buf_v, out_hbm.at[c, pl.ds(s * zr, zr)])

    return seg(feat2, src2, dst)


def _dense(parts, x, w_l, b_l, w_r, do_norm):
    n, d = x.shape
    rb = 1000
    dh = d // NC

    def body(p_ref, x_ref, wl_ref, b_ref, wr_ref, o_ref):
        agg = jnp.concatenate([p_ref[0], p_ref[1]], axis=1)
        dn = (((1,), (1,)), ((), ()))
        y = lax.dot_general(agg, wl_ref[...], dn,
                            preferred_element_type=jnp.float32)
        y = y + lax.dot_general(x_ref[...], wr_ref[...], dn,
                                preferred_element_type=jnp.float32)
        y = y + b_ref[...]
        if do_norm:
            y = jnp.maximum(y, 0.0)
            nrm = jnp.sqrt(jnp.sum(y * y, axis=1, keepdims=True))
            y = y / jnp.maximum(nrm, 1e-12)
        o_ref[...] = y

    return pl.pallas_call(
        body,
        grid=(n // rb,),
        in_specs=[
            pl.BlockSpec((NC, rb, dh), lambda i: (0, i, 0)),
            pl.BlockSpec((rb, d), lambda i: (i, 0)),
            pl.BlockSpec((d, d), lambda i: (0, 0)),
            pl.BlockSpec((1, d), lambda i: (0, 0)),
            pl.BlockSpec((d, d), lambda i: (0, 0)),
        ],
        out_specs=pl.BlockSpec((rb, d), lambda i: (i, 0)),
        out_shape=jax.ShapeDtypeStruct((n, d), jnp.float32),
    )(parts, x, w_l, b_l.reshape(1, d), w_r)


def kernel(x, edge_index, edge_feature, W_l0, b_l0, W_r0, W_l1, b_l1, W_r1):
    n, d = x.shape
    dh = d // NC
    src = edge_index[0]
    dst = edge_index[1]
    src2 = jnp.concatenate([2 * src, 2 * src + 1])

    p0 = _segment_sum_sc(x.reshape(NC * n, dh), src2, dst, n)
    h = _dense(p0, x, W_l0, b_l0, W_r0, do_norm=True)
    p1 = _segment_sum_sc(h.reshape(NC * n, dh), src2, dst, n)
    return _dense(p1, h, W_l1, b_l1, W_r1, do_norm=False)


# speedup vs baseline: 9.2820x; 2.7954x over previous
import functools

import jax
import jax.numpy as jnp
from jax import lax
from jax.experimental import pallas as pl
from jax.experimental.pallas import tpu as pltpu
from jax.experimental.pallas import tpu_sc as plsc

NC = 2
NS = 16
CHUNK = 80


def _segment_sum_sc(feat2, src2, dst3, n):
    dh = feat2.shape[1]
    nchunk = dst3.shape[1]
    npair = nchunk // 2
    zr = -(-n // (NS * 8)) * 8
    n_pad = zr * NS
    zrb = 320
    zra = zr - zrb

    mesh = plsc.VectorSubcoreMesh(core_axis_name="c", subcore_axis_name="s")

    @functools.partial(
        pl.kernel,
        out_type=jax.ShapeDtypeStruct((NC, n_pad, dh), jnp.float32),
        mesh=mesh,
        scratch_types=[
            pltpu.VMEM((nchunk * CHUNK,), jnp.int32),
            pltpu.VMEM((nchunk, CHUNK), jnp.int32),
            pltpu.VMEM((CHUNK, dh), jnp.float32),
            pltpu.VMEM((CHUNK, dh), jnp.float32),
            pltpu.VMEM((zrb, dh), jnp.float32),
            pltpu.VMEM_SHARED((n_pad, dh), jnp.float32),
            pltpu.SemaphoreType.DMA,
            pltpu.SemaphoreType.DMA,
        ],
        compiler_params=pltpu.CompilerParams(use_tc_tiling_on_sc=False),
    )
    def seg(feat_hbm, src_hbm, dst_hbm, out_hbm,
            srcb, dstb, rows_a, rows_b, buf_v, acc_sh, sem_a, sem_b):
        c = lax.axis_index("c")
        s = lax.axis_index("s")

        per_tile = nchunk * CHUNK
        pltpu.sync_copy(src_hbm.at[pl.ds((c * NS + s) * per_tile, per_tile)],
                        srcb)
        pltpu.sync_copy(dst_hbm.at[s], dstb)

        def zrow(r, carry):
            for j in range(dh // 16):
                buf_v[r, pl.ds(j * 16, 16)] = jnp.zeros((16,), jnp.float32)
            return carry
        lax.fori_loop(0, zrb, zrow, 0)
        pltpu.sync_copy(buf_v.at[pl.ds(0, zra)],
                        acc_sh.at[pl.ds(s * zr, zra)])
        pltpu.sync_copy(buf_v, acc_sh.at[pl.ds(s * zr + zra, zrb)])
        plsc.subcore_barrier()

        pltpu.async_copy(feat_hbm.at[srcb.at[pl.ds(0, CHUNK)]], rows_a, sem_a)

        def pair(i, carry):
            g0 = 2 * i
            pltpu.async_copy(feat_hbm.at[srcb.at[pl.ds((g0 + 1) * CHUNK, CHUNK)]], rows_b, sem_b)
            pltpu.make_async_copy(feat_hbm.at[srcb.at[pl.ds(g0 * CHUNK, CHUNK)]],
                                  rows_a, sem_a).wait()
            pltpu.sync_copy(rows_a, acc_sh.at[dstb.at[g0]], add=True)

            @pl.when(i < npair - 1)
            def _():
                pltpu.async_copy(feat_hbm.at[srcb.at[pl.ds((g0 + 2) * CHUNK, CHUNK)]], rows_a, sem_a)

            pltpu.make_async_copy(feat_hbm.at[srcb.at[pl.ds((g0 + 1) * CHUNK, CHUNK)]],
                                  rows_b, sem_b).wait()
            pltpu.sync_copy(rows_b, acc_sh.at[dstb.at[g0 + 1]], add=True)
            return carry
        lax.fori_loop(0, npair, pair, 0)

        plsc.subcore_barrier()
        pltpu.sync_copy(acc_sh.at[pl.ds(s * zr, zra)],
                        buf_v.at[pl.ds(0, zra)])
        pltpu.sync_copy(buf_v.at[pl.ds(0, zra)],
                        out_hbm.at[c, pl.ds(s * zr, zra)])
        pltpu.sync_copy(acc_sh.at[pl.ds(s * zr + zra, zrb)], buf_v)
        pltpu.sync_copy(buf_v, out_hbm.at[c, pl.ds(s * zr + zra, zrb)])

    return seg(feat2, src2, dst3)


def _dense(parts, x, w_l, b_l, w_r, do_norm):
    n, d = x.shape
    rb = 1000
    dh = d // NC

    def body(p_ref, x_ref, wl_ref, b_ref, wr_ref, o_ref):
        agg = jnp.concatenate([p_ref[0], p_ref[1]], axis=1)
        dn = (((1,), (1,)), ((), ()))
        y = lax.dot_general(agg, wl_ref[...], dn,
                            preferred_element_type=jnp.float32)
        y = y + lax.dot_general(x_ref[...], wr_ref[...], dn,
                                preferred_element_type=jnp.float32)
        y = y + b_ref[...]
        if do_norm:
            y = jnp.maximum(y, 0.0)
            nrm = jnp.sqrt(jnp.sum(y * y, axis=1, keepdims=True))
            y = y / jnp.maximum(nrm, 1e-12)
        o_ref[...] = y

    return pl.pallas_call(
        body,
        grid=(n // rb,),
        in_specs=[
            pl.BlockSpec((NC, rb, dh), lambda i: (0, i, 0)),
            pl.BlockSpec((rb, d), lambda i: (i, 0)),
            pl.BlockSpec((d, d), lambda i: (0, 0)),
            pl.BlockSpec((1, d), lambda i: (0, 0)),
            pl.BlockSpec((d, d), lambda i: (0, 0)),
        ],
        out_specs=pl.BlockSpec((rb, d), lambda i: (i, 0)),
        out_shape=jax.ShapeDtypeStruct((n, d), jnp.float32),
    )(parts, x, w_l, b_l.reshape(1, d), w_r)


def kernel(x, edge_index, edge_feature, W_l0, b_l0, W_r0, W_l1, b_l1, W_r1):
    n, d = x.shape
    dh = d // NC
    e = edge_index.shape[1]
    per_tile = e // NS
    nchunk = per_tile // CHUNK
    src = edge_index[0]
    dst = edge_index[1]
    src2 = jnp.concatenate([2 * src, 2 * src + 1])
    dst3 = dst.reshape(NS, nchunk, CHUNK)

    p0 = _segment_sum_sc(x.reshape(NC * n, dh), src2, dst3, n)
    h = _dense(p0, x, W_l0, b_l0, W_r0, do_norm=True)
    p1 = _segment_sum_sc(h.reshape(NC * n, dh), src2, dst3, n)
    return _dense(p1, h, W_l1, b_l1, W_r1, do_norm=False)


# speedup vs baseline: 10.3197x; 1.1118x over previous
import functools

import jax
import jax.numpy as jnp
from jax import lax
from jax.experimental import pallas as pl
from jax.experimental.pallas import tpu as pltpu
from jax.experimental.pallas import tpu_sc as plsc

NC = 2
NS = 16
CHUNK = 80


def _segment_sum_sc(feat2, src2, dst3, n):
    dh = feat2.shape[1]
    nchunk = dst3.shape[1]
    npair = nchunk // 2
    zr = -(-n // (NS * 8)) * 8
    n_pad = zr * NS
    zrb = 320
    zra = zr - zrb

    mesh = plsc.VectorSubcoreMesh(core_axis_name="c", subcore_axis_name="s")

    @functools.partial(
        pl.kernel,
        out_type=jax.ShapeDtypeStruct((NC, n_pad, dh), jnp.float32),
        mesh=mesh,
        scratch_types=[
            pltpu.VMEM((nchunk * CHUNK,), jnp.int32),
            pltpu.VMEM((nchunk, CHUNK), jnp.int32),
            pltpu.VMEM((CHUNK, dh), jnp.float32),
            pltpu.VMEM((CHUNK, dh), jnp.float32),
            pltpu.VMEM((CHUNK, dh), jnp.float32),
            pltpu.VMEM((CHUNK, dh), jnp.float32),
            pltpu.VMEM((zrb, dh), jnp.float32),
            pltpu.VMEM_SHARED((n_pad, dh), jnp.float32),
            pltpu.SemaphoreType.DMA,
            pltpu.SemaphoreType.DMA,
            pltpu.SemaphoreType.DMA,
            pltpu.SemaphoreType.DMA,
            pltpu.SemaphoreType.DMA,
            pltpu.SemaphoreType.DMA,
            pltpu.SemaphoreType.DMA,
            pltpu.SemaphoreType.DMA,
        ],
        compiler_params=pltpu.CompilerParams(use_tc_tiling_on_sc=False),
    )
    def seg(feat_hbm, src_hbm, dst_hbm, out_hbm,
            srcb, dstb, rows_0, rows_1, rows_2, rows_3, buf_v, acc_sh,
            gs_0, gs_1, gs_2, gs_3, ss_0, ss_1, ss_2, ss_3):
        c = lax.axis_index("c")
        s = lax.axis_index("s")

        per_tile = nchunk * CHUNK
        pltpu.sync_copy(src_hbm.at[pl.ds((c * NS + s) * per_tile, per_tile)],
                        srcb)
        pltpu.sync_copy(dst_hbm.at[s], dstb)

        def zrow(r, carry):
            for j in range(dh // 16):
                buf_v[r, pl.ds(j * 16, 16)] = jnp.zeros((16,), jnp.float32)
            return carry
        lax.fori_loop(0, zrb, zrow, 0)
        pltpu.sync_copy(buf_v.at[pl.ds(0, zra)],
                        acc_sh.at[pl.ds(s * zr, zra)])
        pltpu.sync_copy(buf_v, acc_sh.at[pl.ds(s * zr + zra, zrb)])
        plsc.subcore_barrier()

        rows = [rows_0, rows_1, rows_2, rows_3]
        gsem = [gs_0, gs_1, gs_2, gs_3]
        ssem = [ss_0, ss_1, ss_2, ss_3]

        def issue_g(v, b):
            pltpu.async_copy(
                feat_hbm.at[srcb.at[pl.ds(v * CHUNK, CHUNK)]],
                rows[b], gsem[b])

        def wait_g(v, b):
            pltpu.make_async_copy(
                feat_hbm.at[srcb.at[pl.ds(v * CHUNK, CHUNK)]],
                rows[b], gsem[b]).wait()

        def issue_s(v, b):
            pltpu.async_copy(rows[b], acc_sh.at[dstb.at[v]], ssem[b],
                             add=True)

        def wait_s(v, b):
            pltpu.make_async_copy(rows[b], acc_sh.at[dstb.at[v]],
                                  ssem[b]).wait()

        issue_g(0, 0)
        issue_g(1, 1)

        def visit4(i, carry):
            for b in range(4):
                v = 4 * i + b

                @pl.when(v < nchunk)
                def _(v=v, b=b):
                    wait_g(v, b)
                    issue_s(v, b)

                @pl.when(jnp.logical_and(v >= 2, v < nchunk))
                def _(v=v, b=b):
                    wait_s(v - 2, (b - 2) % 4)

                @pl.when(v + 2 < nchunk)
                def _(v=v, b=b):
                    issue_g(v + 2, (b + 2) % 4)
            return carry
        lax.fori_loop(0, (nchunk + 3) // 4, visit4, 0)
        for g in (nchunk - 2, nchunk - 1):
            wait_s(g, g % 4)

        plsc.subcore_barrier()
        pltpu.sync_copy(acc_sh.at[pl.ds(s * zr, zra)],
                        buf_v.at[pl.ds(0, zra)])
        pltpu.sync_copy(buf_v.at[pl.ds(0, zra)],
                        out_hbm.at[c, pl.ds(s * zr, zra)])
        pltpu.sync_copy(acc_sh.at[pl.ds(s * zr + zra, zrb)], buf_v)
        pltpu.sync_copy(buf_v, out_hbm.at[c, pl.ds(s * zr + zra, zrb)])

    return seg(feat2, src2, dst3)


def _dense(parts, x, w_l, b_l, w_r, do_norm):
    n, d = x.shape
    rb = 1000
    dh = d // NC

    def body(p_ref, x_ref, wl_ref, b_ref, wr_ref, o_ref):
        agg = jnp.concatenate([p_ref[0], p_ref[1]], axis=1)
        dn = (((1,), (1,)), ((), ()))
        y = lax.dot_general(agg, wl_ref[...], dn,
                            preferred_element_type=jnp.float32)
        y = y + lax.dot_general(x_ref[...], wr_ref[...], dn,
                                preferred_element_type=jnp.float32)
        y = y + b_ref[...]
        if do_norm:
            y = jnp.maximum(y, 0.0)
            nrm = jnp.sqrt(jnp.sum(y * y, axis=1, keepdims=True))
            y = y / jnp.maximum(nrm, 1e-12)
        o_ref[...] = y

    return pl.pallas_call(
        body,
        grid=(n // rb,),
        in_specs=[
            pl.BlockSpec((NC, rb, dh), lambda i: (0, i, 0)),
            pl.BlockSpec((rb, d), lambda i: (i, 0)),
            pl.BlockSpec((d, d), lambda i: (0, 0)),
            pl.BlockSpec((1, d), lambda i: (0, 0)),
            pl.BlockSpec((d, d), lambda i: (0, 0)),
        ],
        out_specs=pl.BlockSpec((rb, d), lambda i: (i, 0)),
        out_shape=jax.ShapeDtypeStruct((n, d), jnp.float32),
    )(parts, x, w_l, b_l.reshape(1, d), w_r)


def kernel(x, edge_index, edge_feature, W_l0, b_l0, W_r0, W_l1, b_l1, W_r1):
    n, d = x.shape
    dh = d // NC
    e = edge_index.shape[1]
    per_tile = e // NS
    nchunk = per_tile // CHUNK
    src = edge_index[0]
    dst = edge_index[1]
    src2 = jnp.concatenate([2 * src, 2 * src + 1])
    dst3 = dst.reshape(NS, nchunk, CHUNK)

    p0 = _segment_sum_sc(x.reshape(NC * n, dh), src2, dst3, n)
    h = _dense(p0, x, W_l0, b_l0, W_r0, do_norm=True)
    p1 = _segment_sum_sc(h.reshape(NC * n, dh), src2, dst3, n)
    return _dense(p1, h, W_l1, b_l1, W_r1, do_norm=False)
